# Initial kernel scaffold; baseline (speedup 1.0000x reference)
import functools

import jax
import jax.numpy as jnp
from jax import lax
from jax.experimental import pallas as pl
from jax.experimental.pallas import tpu as pltpu
from jax.experimental.pallas import tpu_sc as plsc

_N = 10000
_E = 320000
_NC = 2
_NS = 16
_NW = _NC * _NS
_CH = 80


def _make_agg(n, e, d):
    ept = e // _NW
    n_iter = ept // _CH
    npt = n // _NS
    zr = 125
    mesh = plsc.VectorSubcoreMesh(core_axis_name="c", subcore_axis_name="s")

    @functools.partial(
        pl.kernel,
        mesh=mesh,
        out_type=jax.ShapeDtypeStruct((_NC * n, d), jnp.float32),
        scratch_types=[
            pltpu.VMEM((n_iter, _CH), jnp.int32),
            pltpu.VMEM((n_iter, _CH), jnp.int32),
            pltpu.VMEM((_CH, d), jnp.float32),
            pltpu.VMEM((zr, d), jnp.float32),
            pltpu.VMEM_SHARED((n, d), jnp.float32),
            pltpu.SemaphoreType.DMA,
        ],
    )
    def agg(y_hbm, src_hbm, dst_hbm, out_hbm, sbuf, dbuf, rows, zbuf, acc, sem):
        cid = lax.axis_index("c")
        sid = lax.axis_index("s")
        tid = sid * _NC + cid

        zv = jnp.zeros((16,), jnp.float32)

        def zrow(r, carry):
            for l in range(d // 16):
                zbuf[r, pl.ds(l * 16, 16)] = zv
            return carry

        lax.fori_loop(0, zr, zrow, 0)
        for k in range(npt // zr):
            pltpu.sync_copy(zbuf, acc.at[pl.ds(sid * npt + k * zr, zr)])
        plsc.subcore_barrier()

        base = tid * n_iter
        pltpu.sync_copy(src_hbm.at[pl.ds(base, n_iter)], sbuf)
        pltpu.sync_copy(dst_hbm.at[pl.ds(base, n_iter)], dbuf)

        def body(i, carry):
            pltpu.async_copy(y_hbm.at[sbuf.at[i]], rows, sem).wait()
            pltpu.sync_copy(rows, acc.at[dbuf.at[i]], add=True)
            return carry

        lax.fori_loop(0, n_iter, body, 0)
        plsc.subcore_barrier()
        pltpu.sync_copy(acc.at[pl.ds(sid * npt, npt)],
                        out_hbm.at[pl.ds(cid * n + sid * npt, npt)])

    return agg


def _mm_body(x_ref, w_ref, o_ref):
    o_ref[...] = jnp.dot(x_ref[...], w_ref[...],
                         preferred_element_type=jnp.float32)


def _matmul(x, w):
    return pl.pallas_call(
        _mm_body,
        out_shape=jax.ShapeDtypeStruct((x.shape[0], w.shape[1]), jnp.float32),
    )(x, w)


def _combine_mm_body(y_ref, p_ref, b_ref, s_ref, w_ref, o_ref):
    h = s_ref[...] * y_ref[...] + p_ref[0] + p_ref[1] + b_ref[...]
    h = jnp.maximum(h, 0.0)
    o_ref[...] = jnp.dot(h, w_ref[...], preferred_element_type=jnp.float32)


def _combine_mm(y, p, b, s, w):
    return pl.pallas_call(
        _combine_mm_body,
        out_shape=jax.ShapeDtypeStruct((y.shape[0], w.shape[1]), jnp.float32),
    )(y, p, b, s, w)


def _final_body(y_ref, p_ref, b_ref, s_ref, lo_ref, pr_ref):
    logits = s_ref[...] * y_ref[...] + p_ref[0] + p_ref[1] + b_ref[...]
    lo_ref[...] = logits
    m = jnp.max(logits, axis=-1, keepdims=True)
    ex = jnp.exp(logits - m)
    pr_ref[...] = ex / jnp.sum(ex, axis=-1, keepdims=True)


def _final(y, p, b, s):
    n, d = y.shape
    return pl.pallas_call(
        _final_body,
        out_shape=(jax.ShapeDtypeStruct((n, d), jnp.float32),
                   jax.ShapeDtypeStruct((n, d), jnp.float32)),
    )(y, p, b, s)


@jax.jit
def kernel(x, edge_index, W1, b1, eps1, W2, b2, eps2, W3, b3, eps3):
    src = edge_index[0].reshape(_E // _CH, _CH)
    dst = edge_index[1].reshape(_E // _CH, _CH)
    s1 = jnp.reshape(1.0 + eps1, (1, 1))
    s2 = jnp.reshape(1.0 + eps2, (1, 1))
    s3 = jnp.reshape(1.0 + eps3, (1, 1))

    agg128 = _make_agg(_N, _E, 128)
    agg64 = _make_agg(_N, _E, 64)

    y1 = _matmul(x, W1)
    p1 = agg128(y1, src, dst).reshape(_NC, _N, 128)
    y2 = _combine_mm(y1, p1, b1.reshape(1, -1), s1, W2)
    p2 = agg128(y2, src, dst).reshape(_NC, _N, 128)
    y3 = _combine_mm(y2, p2, b2.reshape(1, -1), s2, W3)
    p3 = agg64(y3, src, dst).reshape(_NC, _N, 64)
    logits, probs = _final(y3, p3, b3.reshape(1, -1), s3)
    return (logits, probs)


# speedup vs baseline: 5.4922x; 5.4922x over previous
import functools

import jax
import jax.numpy as jnp
from jax import lax
from jax.experimental import pallas as pl
from jax.experimental.pallas import tpu as pltpu
from jax.experimental.pallas import tpu_sc as plsc

_N = 10000
_NP = 10240
_E = 320000
_NC = 2
_NS = 16
_NW = _NC * _NS
_CH = 128


def _make_agg(d):
    n_chunks = _E // _CH
    n_iter = (n_chunks + _NW - 1) // _NW
    npt = _NP // _NS
    zr = 128
    mesh = plsc.VectorSubcoreMesh(core_axis_name="c", subcore_axis_name="s")

    @functools.partial(
        pl.kernel,
        mesh=mesh,
        out_type=jax.ShapeDtypeStruct((_NC * _NP, d), jnp.float32),
        scratch_types=[
            pltpu.VMEM((_CH,), jnp.int32),
            pltpu.VMEM((_CH,), jnp.int32),
            pltpu.VMEM((_CH, d), jnp.float32),
            pltpu.VMEM((zr, d), jnp.float32),
            pltpu.VMEM_SHARED((_NP, d), jnp.float32),
            pltpu.SemaphoreType.DMA,
        ],
    )
    def agg(y_hbm, src_hbm, dst_hbm, out_hbm, sidx, didx, rows, zbuf, acc, sem):
        cid = lax.axis_index("c")
        sid = lax.axis_index("s")
        tid = sid * _NC + cid

        zv = jnp.zeros((16,), jnp.float32)

        def zrow(r, carry):
            for l in range(d // 16):
                zbuf[r, pl.ds(l * 16, 16)] = zv
            return carry

        lax.fori_loop(0, zr, zrow, 0)
        for k in range(npt // zr):
            pltpu.sync_copy(zbuf, acc.at[pl.ds(sid * npt + k * zr, zr)])
        plsc.subcore_barrier()

        def body(i, carry):
            c = i * _NW + tid

            @pl.when(c < n_chunks)
            def _():
                base = c * _CH
                pltpu.sync_copy(src_hbm.at[pl.ds(base, _CH)], sidx)
                pltpu.sync_copy(dst_hbm.at[pl.ds(base, _CH)], didx)
                pltpu.async_copy(y_hbm.at[sidx], rows, sem).wait()
                pltpu.sync_copy(rows, acc.at[didx], add=True)

            return carry

        lax.fori_loop(0, n_iter, body, 0)
        plsc.subcore_barrier()
        pltpu.sync_copy(acc.at[pl.ds(sid * npt, npt)],
                        out_hbm.at[pl.ds(cid * _NP + sid * npt, npt)])

    return agg


def _mm_body(x_ref, w_ref, o_ref):
    o_ref[...] = jnp.dot(x_ref[...], w_ref[...],
                         preferred_element_type=jnp.float32)


def _matmul(x, w):
    return pl.pallas_call(
        _mm_body,
        out_shape=jax.ShapeDtypeStruct((x.shape[0], w.shape[1]), jnp.float32),
    )(x, w)


def _combine_mm_body(y_ref, p_ref, b_ref, s_ref, w_ref, o_ref):
    h = s_ref[...] * y_ref[...] + p_ref[0] + p_ref[1] + b_ref[...]
    h = jnp.maximum(h, 0.0)
    o_ref[...] = jnp.dot(h, w_ref[...], preferred_element_type=jnp.float32)


def _combine_mm(y, p, b, s, w):
    return pl.pallas_call(
        _combine_mm_body,
        out_shape=jax.ShapeDtypeStruct((y.shape[0], w.shape[1]), jnp.float32),
    )(y, p, b, s, w)


def _combine_body(y_ref, p_ref, b_ref, s_ref, o_ref):
    h = s_ref[...] * y_ref[...] + p_ref[0] + p_ref[1] + b_ref[...]
    o_ref[...] = jnp.maximum(h, 0.0)


def _combine(y, p, b, s):
    return pl.pallas_call(
        _combine_body,
        out_shape=jax.ShapeDtypeStruct(y.shape, jnp.float32),
    )(y, p, b, s)


def _final_body(h_ref, p_ref, b_ref, s_ref, w_ref, lo_ref, pr_ref):
    z = s_ref[...] * h_ref[...] + p_ref[0] + p_ref[1]
    logits = jnp.dot(z, w_ref[...], preferred_element_type=jnp.float32)
    logits = jnp.maximum(logits + b_ref[...], 0.0)
    lo_ref[...] = logits
    m = jnp.max(logits, axis=-1, keepdims=True)
    ex = jnp.exp(logits - m)
    pr_ref[...] = ex / jnp.sum(ex, axis=-1, keepdims=True)


def _final(h, p, b, s, w):
    n = h.shape[0]
    d = w.shape[1]
    return pl.pallas_call(
        _final_body,
        out_shape=(jax.ShapeDtypeStruct((n, d), jnp.float32),
                   jax.ShapeDtypeStruct((n, d), jnp.float32)),
    )(h, p, b, s, w)


@jax.jit
def kernel(x, edge_index, W1, b1, eps1, W2, b2, eps2, W3, b3, eps3):
    src = edge_index[0]
    dst = edge_index[1]
    s1 = jnp.reshape(1.0 + eps1, (1, 1))
    s2 = jnp.reshape(1.0 + eps2, (1, 1))
    s3 = jnp.reshape(1.0 + eps3, (1, 1))

    agg128 = _make_agg(128)

    y1 = _matmul(x, W1)
    p1 = agg128(y1, src, dst).reshape(_NC, _NP, 128)[:, :_N, :]
    y2 = _combine_mm(y1, p1, b1.reshape(1, -1), s1, W2)
    p2 = agg128(y2, src, dst).reshape(_NC, _NP, 128)[:, :_N, :]
    h2 = _combine(y2, p2, b2.reshape(1, -1), s2)
    p3 = agg128(h2, src, dst).reshape(_NC, _NP, 128)[:, :_N, :]
    logits, probs = _final(h2, p3, b3.reshape(1, -1), s3, W3)
    return (logits, probs)
